# Initial kernel scaffold; baseline (speedup 1.0000x reference)
import functools

import jax
import jax.numpy as jnp
from jax import lax
from jax.experimental import pallas as pl
from jax.experimental.pallas import tpu as pltpu
from jax.experimental.pallas import tpu_sc as plsc

_EPS = 1e-12
_K = 128


def _build_sc_gather(vocab, hid, tok, nc, ns):
    nw = nc * ns
    per_w = tok // nw
    nj = per_w // _K
    assert per_w % _K == 0 and nj % 2 == 0

    mesh = plsc.VectorSubcoreMesh(core_axis_name="c", subcore_axis_name="s")

    @functools.partial(
        pl.kernel,
        mesh=mesh,
        out_type=jax.ShapeDtypeStruct((tok, hid), jnp.float32),
        scratch_types=[
            pltpu.VMEM((nj, _K), jnp.int32),
            pltpu.VMEM((_K, hid), jnp.float32),
            pltpu.VMEM((_K, hid), jnp.float32),
            pltpu.SemaphoreType.DMA,
            pltpu.SemaphoreType.DMA,
        ],
    )
    def sc_gather(table, idx, out, idx_v, buf0, buf1, g0, g1):
        wid = lax.axis_index("s") * nc + lax.axis_index("c")
        base = wid * per_w
        pltpu.sync_copy(idx.at[wid], idx_v)
        bufs = (buf0, buf1)
        gsems = (g0, g1)
        for b in range(2):
            pltpu.make_async_copy(table.at[idx_v.at[b]], bufs[b], gsems[b]).start()

        def step(i, carry):
            for b in range(2):
                j = i * 2 + b
                pltpu.make_async_copy(table.at[idx_v.at[j]], bufs[b], gsems[b]).wait()
                pltpu.sync_copy(bufs[b], out.at[pl.ds(base + j * _K, _K)])

                @pl.when(j + 2 < nj)
                def _():
                    pltpu.make_async_copy(
                        table.at[idx_v.at[j + 2]], bufs[b], gsems[b]
                    ).start()

            return carry

        lax.fori_loop(0, nj // 2, step, 0)

    return sc_gather, nw, nj


def _ln_body(x_ref, sid_ref, pos_ref, seg_ref, w_ref, b_ref, o_ref):
    x = x_ref[...]
    sid = sid_ref[...]
    seg = jnp.where(
        (sid == 0)[..., None], seg_ref[0][None, None, :], seg_ref[1][None, None, :]
    )
    e = x + pos_ref[...][None] + seg
    mu = jnp.mean(e, axis=-1, keepdims=True)
    d = e - mu
    var = jnp.mean(d * d, axis=-1, keepdims=True)
    inv = lax.rsqrt(var + _EPS)
    o_ref[...] = d * inv * w_ref[0][None, None, :] + b_ref[0][None, None, :]


def kernel(token_ids, segment_ids, word_emb, pos_emb, seg_emb, ln_w, ln_b):
    bsz, seq = token_ids.shape
    vocab, hid = word_emb.shape
    tok = bsz * seq

    info = plsc.get_sparse_core_info()
    nc, ns = info.num_cores, info.num_subcores
    sc_gather, nw, nj = _build_sc_gather(vocab, hid, tok, nc, ns)

    idx = token_ids.astype(jnp.int32).reshape(nw, nj, _K)
    gathered = sc_gather(word_emb, idx).reshape(bsz, seq, hid)

    bb = 64
    assert bsz % bb == 0
    out = pl.pallas_call(
        _ln_body,
        grid=(bsz // bb,),
        in_specs=[
            pl.BlockSpec((bb, seq, hid), lambda i: (i, 0, 0)),
            pl.BlockSpec((bb, seq), lambda i: (i, 0)),
            pl.BlockSpec((seq, hid), lambda i: (0, 0)),
            pl.BlockSpec((2, hid), lambda i: (0, 0)),
            pl.BlockSpec((1, hid), lambda i: (0, 0)),
            pl.BlockSpec((1, hid), lambda i: (0, 0)),
        ],
        out_specs=pl.BlockSpec((bb, seq, hid), lambda i: (i, 0, 0)),
        out_shape=jax.ShapeDtypeStruct((bsz, seq, hid), jnp.float32),
        compiler_params=pltpu.CompilerParams(dimension_semantics=("parallel",)),
    )(
        gathered,
        segment_ids.astype(jnp.int32),
        pos_emb[:seq],
        seg_emb,
        ln_w.reshape(1, hid),
        ln_b.reshape(1, hid),
    )
    return out


# speedup vs baseline: 5.2679x; 5.2679x over previous
import functools

import jax
import jax.numpy as jnp
from jax import lax
from jax.experimental import pallas as pl
from jax.experimental.pallas import tpu as pltpu
from jax.experimental.pallas import tpu_sc as plsc

_EPS = 1e-12
_K = 128


def _build_sc_gather(vocab, hid, tok, nc, ns):
    nw = nc * ns
    per_w = tok // nw
    nj = per_w // _K
    assert per_w % _K == 0 and nj % 2 == 0

    mesh = plsc.VectorSubcoreMesh(core_axis_name="c", subcore_axis_name="s")

    @functools.partial(
        pl.kernel,
        mesh=mesh,
        out_type=jax.ShapeDtypeStruct((tok, hid), jnp.float32),
        scratch_types=[
            pltpu.VMEM((nj, _K), jnp.int32),
            pltpu.VMEM((_K, hid), jnp.float32),
            pltpu.VMEM((_K, hid), jnp.float32),
            pltpu.SemaphoreType.DMA,
            pltpu.SemaphoreType.DMA,
        ],
    )
    def sc_gather(table, idx, out, idx_v, buf0, buf1, g0, g1):
        wid = lax.axis_index("s") * nc + lax.axis_index("c")
        base = wid * per_w
        pltpu.sync_copy(idx.at[wid], idx_v)
        bufs = (buf0, buf1)
        gsems = (g0, g1)
        for b in range(2):
            pltpu.make_async_copy(table.at[idx_v.at[b]], bufs[b], gsems[b]).start()

        def step(i, carry):
            for b in range(2):
                j = i * 2 + b
                pltpu.make_async_copy(table.at[idx_v.at[j]], bufs[b], gsems[b]).wait()
                pltpu.sync_copy(bufs[b], out.at[pl.ds(base + j * _K, _K)])

                @pl.when(j + 2 < nj)
                def _():
                    pltpu.make_async_copy(
                        table.at[idx_v.at[j + 2]], bufs[b], gsems[b]
                    ).start()

            return carry

        lax.fori_loop(0, nj // 2, step, 0)

    return sc_gather, nw, nj


def _ln_body(x_ref, sid_ref, pos_ref, seg_ref, w_ref, b_ref, o_ref):
    x = x_ref[...]
    sid = sid_ref[...]
    seg = jnp.where(sid == 0, seg_ref[0][None, :], seg_ref[1][None, :])
    e = x + pos_ref[...] + seg
    mu = jnp.mean(e, axis=-1, keepdims=True)
    d = e - mu
    var = jnp.mean(d * d, axis=-1, keepdims=True)
    inv = lax.rsqrt(var + _EPS)
    o_ref[...] = d * inv * w_ref[0][None, :] + b_ref[0][None, :]


def kernel(token_ids, segment_ids, word_emb, pos_emb, seg_emb, ln_w, ln_b):
    bsz, seq = token_ids.shape
    vocab, hid = word_emb.shape
    tok = bsz * seq

    info = plsc.get_sparse_core_info()
    nc, ns = info.num_cores, info.num_subcores
    sc_gather, nw, nj = _build_sc_gather(vocab, hid, tok, nc, ns)

    idx = token_ids.astype(jnp.int32).reshape(nw, nj, _K)
    gathered = sc_gather(word_emb, idx)

    tb = 8 * seq
    assert tok % tb == 0
    pos_tile = jnp.tile(pos_emb[:seq], (tb // seq, 1))
    out = pl.pallas_call(
        _ln_body,
        grid=(tok // tb,),
        in_specs=[
            pl.BlockSpec((tb, hid), lambda i: (i, 0)),
            pl.BlockSpec((tb, 1), lambda i: (i, 0)),
            pl.BlockSpec((tb, hid), lambda i: (0, 0)),
            pl.BlockSpec((2, hid), lambda i: (0, 0)),
            pl.BlockSpec((1, hid), lambda i: (0, 0)),
            pl.BlockSpec((1, hid), lambda i: (0, 0)),
        ],
        out_specs=pl.BlockSpec((tb, hid), lambda i: (i, 0)),
        out_shape=jax.ShapeDtypeStruct((tok, hid), jnp.float32),
        compiler_params=pltpu.CompilerParams(dimension_semantics=("parallel",)),
    )(
        gathered,
        segment_ids.astype(jnp.int32).reshape(tok, 1),
        pos_tile,
        seg_emb,
        ln_w.reshape(1, hid),
        ln_b.reshape(1, hid),
    )
    return out.reshape(bsz, seq, hid)


# speedup vs baseline: 5.3455x; 1.0147x over previous
import functools

import jax
import jax.numpy as jnp
from jax import lax
from jax.experimental import pallas as pl
from jax.experimental.pallas import tpu as pltpu
from jax.experimental.pallas import tpu_sc as plsc

_EPS = 1e-12
_K = 128
_NL = 16
_NBUF = 3


def _build_sc_gather(vocab, hid, tok, nps, nc, ns):
    nw = nc * ns
    per_w = tok // nw
    nj = per_w // _K
    assert per_w % _K == 0 and hid % _NL == 0 and nj >= _NBUF
    nfull = (nj // _NBUF) * _NBUF

    mesh = plsc.VectorSubcoreMesh(core_axis_name="c", subcore_axis_name="s")

    @functools.partial(
        pl.kernel,
        mesh=mesh,
        out_type=jax.ShapeDtypeStruct((tok, hid), jnp.float32),
        scratch_types=[
            pltpu.VMEM((nj, _K), jnp.int32),
            pltpu.VMEM((per_w,), jnp.int32),
            pltpu.VMEM((nps * hid,), jnp.float32),
            [pltpu.VMEM((_K, hid), jnp.float32) for _ in range(_NBUF)],
            [pltpu.SemaphoreType.DMA for _ in range(_NBUF)],
            [pltpu.SemaphoreType.DMA for _ in range(_NBUF)],
        ],
    )
    def sc_gather(table, idx, psidx, ps, out, idx_v, psidx_v, ps_v, bufs,
                  gsems, osems):
        wid = lax.axis_index("s") * nc + lax.axis_index("c")
        base = wid * per_w
        pltpu.sync_copy(idx.at[wid], idx_v)
        pltpu.sync_copy(psidx.at[wid], psidx_v)
        pltpu.sync_copy(ps, ps_v)
        lanes = lax.iota(jnp.int32, _NL)

        def start_gather(j, b):
            pltpu.make_async_copy(table.at[idx_v.at[j]], bufs[b], gsems[b]).start()

        def wait_gather(j, b):
            pltpu.make_async_copy(table.at[idx_v.at[j]], bufs[b], gsems[b]).wait()

        def start_out(j, b):
            pltpu.make_async_copy(
                bufs[b], out.at[pl.ds(base + j * _K, _K)], osems[b]
            ).start()

        def wait_out(b):
            pltpu.make_async_copy(
                bufs[b], out.at[pl.ds(base, _K)], osems[b]
            ).wait()

        def ps_adds(j, b):
            buf = bufs[b]
            jbase = j * _K

            def group_add(g, c):
                riv = psidx_v[pl.ds(jbase + g * _NL, _NL)] * hid
                for u in range(_NL):
                    rr = g * _NL + u
                    rowbase = riv[u]
                    for k in range(hid // _NL):
                        v = ps_v[pl.ds(rowbase + k * _NL, _NL)]
                        plsc.addupdate(buf.at[rr, pl.ds(k * _NL, _NL)], v)
                return c

            lax.fori_loop(0, _K // _NL, group_add, 0)

        def half_step(j, b):
            wait_gather(j, b)
            ps_adds(j, b)
            start_out(j, b)
            nb = (b + _NBUF - 1) % _NBUF

            @pl.when(jnp.logical_and(j >= 1, j + _NBUF - 1 < nj))
            def _():
                wait_out(nb)

            @pl.when(j + _NBUF - 1 < nj)
            def _():
                start_gather(j + _NBUF - 1, nb)

        for b in range(_NBUF - 1):
            start_gather(b, b)

        def step(i, carry):
            for b in range(_NBUF):
                half_step(i * _NBUF + b, b)
            return carry

        lax.fori_loop(0, nfull // _NBUF, step, 0)
        for j in range(nfull, nj):
            half_step(j, j % _NBUF)
        for j in range(nj - _NBUF, nj):
            wait_out(j % _NBUF)

    return sc_gather, nw, nj


def _ln_body(x_ref, w_ref, b_ref, o_ref):
    x = x_ref[...]
    mu = jnp.mean(x, axis=-1, keepdims=True)
    d = x - mu
    var = jnp.mean(d * d, axis=-1, keepdims=True)
    inv = lax.rsqrt(var + _EPS)
    o_ref[...] = d * inv * w_ref[0][None, :] + b_ref[0][None, :]


def kernel(token_ids, segment_ids, word_emb, pos_emb, seg_emb, ln_w, ln_b):
    bsz, seq = token_ids.shape
    vocab, hid = word_emb.shape
    nseg = seg_emb.shape[0]
    tok = bsz * seq
    nps = nseg * seq

    info = plsc.get_sparse_core_info()
    nc, ns = info.num_cores, info.num_subcores
    sc_gather, nw, nj = _build_sc_gather(vocab, hid, tok, nps, nc, ns)

    idx = token_ids.astype(jnp.int32).reshape(nw, nj, _K)
    ps = (pos_emb[:seq][None, :, :] + seg_emb[:, None, :]).reshape(nps, hid)
    psidx = (
        segment_ids.astype(jnp.int32) * seq + jnp.arange(seq, dtype=jnp.int32)[None, :]
    ).reshape(nw, tok // nw)
    summed = sc_gather(word_emb, idx, psidx, ps.reshape(-1))

    tb = 12800
    assert tok % tb == 0
    out = pl.pallas_call(
        _ln_body,
        grid=(tok // tb,),
        in_specs=[
            pl.BlockSpec((tb, hid), lambda i: (i, 0)),
            pl.BlockSpec((1, hid), lambda i: (0, 0)),
            pl.BlockSpec((1, hid), lambda i: (0, 0)),
        ],
        out_specs=pl.BlockSpec((tb, hid), lambda i: (i, 0)),
        out_shape=jax.ShapeDtypeStruct((tok, hid), jnp.float32),
        compiler_params=pltpu.CompilerParams(dimension_semantics=("parallel",)),
    )(summed, ln_w.reshape(1, hid), ln_b.reshape(1, hid))
    return out.reshape(bsz, seq, hid)


# speedup vs baseline: 5.9444x; 1.1121x over previous
import functools

import jax
import jax.numpy as jnp
from jax import lax
from jax.experimental import pallas as pl
from jax.experimental.pallas import tpu as pltpu
from jax.experimental.pallas import tpu_sc as plsc

_EPS = 1e-12
_K = 128
_NBUF = 3


def _build_sc_gather(vocab, hid, tok, nps, nc, ns):
    nw = nc * ns
    per_w = tok // nw
    nj = per_w // _K
    assert per_w % _K == 0 and nj >= _NBUF
    nfull = (nj // _NBUF) * _NBUF

    mesh = plsc.VectorSubcoreMesh(core_axis_name="c", subcore_axis_name="s")

    @functools.partial(
        pl.kernel,
        mesh=mesh,
        out_type=jax.ShapeDtypeStruct((tok, hid), jnp.float32),
        scratch_types=[
            pltpu.VMEM((nj, _K), jnp.int32),
            pltpu.VMEM((nj, _K), jnp.int32),
            [pltpu.VMEM((_K, hid), jnp.float32) for _ in range(_NBUF)],
            [pltpu.SemaphoreType.DMA for _ in range(_NBUF)],
            [pltpu.SemaphoreType.DMA for _ in range(_NBUF)],
            [pltpu.SemaphoreType.DMA for _ in range(_NBUF)],
        ],
    )
    def sc_gather(table, idx, psidx, ps, out, idx_v, psidx_v, bufs,
                  gsems, asems, osems):
        wid = lax.axis_index("s") * nc + lax.axis_index("c")
        base = wid * per_w
        pltpu.sync_copy(idx.at[wid], idx_v)
        pltpu.sync_copy(psidx.at[wid], psidx_v)

        def start_gather(j, b):
            pltpu.make_async_copy(table.at[idx_v.at[j]], bufs[b], gsems[b]).start()

        def wait_gather(b):
            pltpu.make_async_copy(table.at[idx_v.at[0]], bufs[b], gsems[b]).wait()

        def start_out(j, b):
            pltpu.make_async_copy(
                bufs[b], out.at[pl.ds(base + j * _K, _K)], osems[b]
            ).start()

        def wait_out(b):
            pltpu.make_async_copy(
                bufs[b], out.at[pl.ds(base, _K)], osems[b]
            ).wait()

        def ps_add(j, b):
            cp = pltpu.make_async_copy(ps.at[psidx_v.at[j]], bufs[b], asems[b])
            cp.start(add=True)
            cp.wait()

        def half_step(j, b):
            wait_gather(b)
            ps_add(j, b)
            start_out(j, b)
            nb = (b + _NBUF - 1) % _NBUF

            @pl.when(jnp.logical_and(j >= 1, j + _NBUF - 1 < nj))
            def _():
                wait_out(nb)

            @pl.when(j + _NBUF - 1 < nj)
            def _():
                start_gather(j + _NBUF - 1, nb)

        for b in range(_NBUF - 1):
            start_gather(b, b)

        def step(i, carry):
            for b in range(_NBUF):
                half_step(i * _NBUF + b, b)
            return carry

        lax.fori_loop(0, nfull // _NBUF, step, 0)
        for j in range(nfull, nj):
            half_step(j, j % _NBUF)
        for j in range(nj - _NBUF, nj):
            wait_out(j % _NBUF)

    return sc_gather, nw, nj


def _ln_body(x_ref, w_ref, b_ref, o_ref):
    x = x_ref[...]
    mu = jnp.mean(x, axis=-1, keepdims=True)
    d = x - mu
    var = jnp.mean(d * d, axis=-1, keepdims=True)
    inv = lax.rsqrt(var + _EPS)
    o_ref[...] = d * inv * w_ref[0][None, :] + b_ref[0][None, :]


def kernel(token_ids, segment_ids, word_emb, pos_emb, seg_emb, ln_w, ln_b):
    bsz, seq = token_ids.shape
    vocab, hid = word_emb.shape
    nseg = seg_emb.shape[0]
    tok = bsz * seq
    nps = nseg * seq

    info = plsc.get_sparse_core_info()
    nc, ns = info.num_cores, info.num_subcores
    sc_gather, nw, nj = _build_sc_gather(vocab, hid, tok, nps, nc, ns)

    idx = token_ids.astype(jnp.int32).reshape(nw, nj, _K)
    ps = (pos_emb[:seq][None, :, :] + seg_emb[:, None, :]).reshape(nps, hid)
    psidx = (
        segment_ids.astype(jnp.int32) * seq + jnp.arange(seq, dtype=jnp.int32)[None, :]
    ).reshape(nw, nj, _K)
    summed = sc_gather(word_emb, idx, psidx, ps)

    tb = 12800
    assert tok % tb == 0
    out = pl.pallas_call(
        _ln_body,
        grid=(tok // tb,),
        in_specs=[
            pl.BlockSpec((tb, hid), lambda i: (i, 0)),
            pl.BlockSpec((1, hid), lambda i: (0, 0)),
            pl.BlockSpec((1, hid), lambda i: (0, 0)),
        ],
        out_specs=pl.BlockSpec((tb, hid), lambda i: (i, 0)),
        out_shape=jax.ShapeDtypeStruct((tok, hid), jnp.float32),
        compiler_params=pltpu.CompilerParams(dimension_semantics=("parallel",)),
    )(summed, ln_w.reshape(1, hid), ln_b.reshape(1, hid))
    return out.reshape(bsz, seq, hid)


# speedup vs baseline: 5.9841x; 1.0067x over previous
import functools

import jax
import jax.numpy as jnp
from jax import lax
from jax.experimental import pallas as pl
from jax.experimental.pallas import tpu as pltpu
from jax.experimental.pallas import tpu_sc as plsc

_EPS = 1e-12
_K = 128
_NBUF = 5


def _build_sc_gather(vocab, hid, tok, nps, nc, ns):
    nw = nc * ns
    per_w = tok // nw
    nj = per_w // _K
    assert per_w % _K == 0 and nj >= _NBUF
    nfull = (nj // _NBUF) * _NBUF

    mesh = plsc.VectorSubcoreMesh(core_axis_name="c", subcore_axis_name="s")

    @functools.partial(
        pl.kernel,
        mesh=mesh,
        out_type=jax.ShapeDtypeStruct((tok, hid), jnp.float32),
        scratch_types=[
            pltpu.VMEM((nj, _K), jnp.int32),
            pltpu.VMEM((nj, _K), jnp.int32),
            [pltpu.VMEM((_K, hid), jnp.float32) for _ in range(_NBUF)],
            [pltpu.SemaphoreType.DMA for _ in range(_NBUF)],
            [pltpu.SemaphoreType.DMA for _ in range(_NBUF)],
            [pltpu.SemaphoreType.DMA for _ in range(_NBUF)],
        ],
    )
    def sc_gather(table, idx, psidx, ps, out, idx_v, psidx_v, bufs,
                  gsems, asems, osems):
        wid = lax.axis_index("s") * nc + lax.axis_index("c")
        base = wid * per_w
        pltpu.sync_copy(idx.at[wid], idx_v)
        pltpu.sync_copy(psidx.at[wid], psidx_v)

        def start_gather(j, b):
            pltpu.make_async_copy(table.at[idx_v.at[j]], bufs[b], gsems[b]).start()

        def wait_gather(b):
            pltpu.make_async_copy(table.at[idx_v.at[0]], bufs[b], gsems[b]).wait()

        def start_out(j, b):
            pltpu.make_async_copy(
                bufs[b], out.at[pl.ds(base + j * _K, _K)], osems[b]
            ).start()

        def wait_out(b):
            pltpu.make_async_copy(
                bufs[b], out.at[pl.ds(base, _K)], osems[b]
            ).wait()

        def half_step(j, b):
            wait_gather(b)
            cp = pltpu.make_async_copy(ps.at[psidx_v.at[j]], bufs[b], asems[b])
            cp.start(add=True)
            nb = (b + _NBUF - 1) % _NBUF

            @pl.when(jnp.logical_and(j >= 1, j + _NBUF - 1 < nj))
            def _():
                wait_out(nb)

            @pl.when(j + _NBUF - 1 < nj)
            def _():
                start_gather(j + _NBUF - 1, nb)

            cp.wait()
            start_out(j, b)

        for b in range(_NBUF - 1):
            start_gather(b, b)

        def step(i, carry):
            for b in range(_NBUF):
                half_step(i * _NBUF + b, b)
            return carry

        lax.fori_loop(0, nfull // _NBUF, step, 0)
        for j in range(nfull, nj):
            half_step(j, j % _NBUF)
        for j in range(nj - _NBUF, nj):
            wait_out(j % _NBUF)

    return sc_gather, nw, nj


def _ln_body(x_ref, w_ref, b_ref, o_ref):
    x = x_ref[...]
    mu = jnp.mean(x, axis=-1, keepdims=True)
    d = x - mu
    var = jnp.mean(d * d, axis=-1, keepdims=True)
    inv = lax.rsqrt(var + _EPS)
    o_ref[...] = d * inv * w_ref[0][None, :] + b_ref[0][None, :]


def kernel(token_ids, segment_ids, word_emb, pos_emb, seg_emb, ln_w, ln_b):
    bsz, seq = token_ids.shape
    vocab, hid = word_emb.shape
    nseg = seg_emb.shape[0]
    tok = bsz * seq
    nps = nseg * seq

    info = plsc.get_sparse_core_info()
    nc, ns = info.num_cores, info.num_subcores
    sc_gather, nw, nj = _build_sc_gather(vocab, hid, tok, nps, nc, ns)

    idx = token_ids.astype(jnp.int32).reshape(nw, nj, _K)
    ps = (pos_emb[:seq][None, :, :] + seg_emb[:, None, :]).reshape(nps, hid)
    psidx = (
        segment_ids.astype(jnp.int32) * seq + jnp.arange(seq, dtype=jnp.int32)[None, :]
    ).reshape(nw, nj, _K)
    summed = sc_gather(word_emb, idx, psidx, ps)

    tb = 20480
    assert tok % tb == 0
    out = pl.pallas_call(
        _ln_body,
        grid=(tok // tb,),
        in_specs=[
            pl.BlockSpec((tb, hid), lambda i: (i, 0)),
            pl.BlockSpec((1, hid), lambda i: (0, 0)),
            pl.BlockSpec((1, hid), lambda i: (0, 0)),
        ],
        out_specs=pl.BlockSpec((tb, hid), lambda i: (i, 0)),
        out_shape=jax.ShapeDtypeStruct((tok, hid), jnp.float32),
        compiler_params=pltpu.CompilerParams(dimension_semantics=("parallel",)),
    )(summed, ln_w.reshape(1, hid), ln_b.reshape(1, hid))
    return out.reshape(bsz, seq, hid)


# speedup vs baseline: 6.0259x; 1.0070x over previous
import functools

import jax
import jax.numpy as jnp
from jax import lax
from jax.experimental import pallas as pl
from jax.experimental.pallas import tpu as pltpu
from jax.experimental.pallas import tpu_sc as plsc

_EPS = 1e-12
_K = 128
_NBUF = 5


def _build_sc_gather(vocab, hid, tok, nps, nc, ns):
    nw = nc * ns
    per_w = tok // nw
    nj = per_w // _K
    assert per_w % _K == 0 and nj >= _NBUF
    nfull = (nj // _NBUF) * _NBUF

    mesh = plsc.VectorSubcoreMesh(core_axis_name="c", subcore_axis_name="s")

    @functools.partial(
        pl.kernel,
        mesh=mesh,
        out_type=jax.ShapeDtypeStruct((tok, hid), jnp.float32),
        scratch_types=[
            pltpu.VMEM((nj, _K), jnp.int32),
            pltpu.VMEM((nj, _K), jnp.int32),
            [pltpu.VMEM((_K, hid), jnp.float32) for _ in range(_NBUF)],
            [pltpu.SemaphoreType.DMA for _ in range(_NBUF)],
            [pltpu.SemaphoreType.DMA for _ in range(_NBUF)],
            [pltpu.SemaphoreType.DMA for _ in range(_NBUF)],
        ],
    )
    def sc_gather(table, idx, psidx, ps, out, idx_v, psidx_v, bufs,
                  gsems, asems, osems):
        wid = lax.axis_index("s") * nc + lax.axis_index("c")
        base = wid * per_w
        pltpu.sync_copy(idx.at[wid], idx_v)
        pltpu.sync_copy(psidx.at[wid], psidx_v)

        def start_gather(j, b):
            pltpu.make_async_copy(table.at[idx_v.at[j]], bufs[b], gsems[b]).start()

        def wait_gather(b):
            pltpu.make_async_copy(table.at[idx_v.at[0]], bufs[b], gsems[b]).wait()

        def start_out(j, b):
            pltpu.make_async_copy(
                bufs[b], out.at[pl.ds(base + j * _K, _K)], osems[b]
            ).start()

        def wait_out(b):
            pltpu.make_async_copy(
                bufs[b], out.at[pl.ds(base, _K)], osems[b]
            ).wait()

        def start_add(j, b):
            pltpu.make_async_copy(
                ps.at[psidx_v.at[j]], bufs[b], asems[b]
            ).start(add=True)

        def wait_add(b):
            pltpu.make_async_copy(
                ps.at[psidx_v.at[0]], bufs[b], asems[b]
            ).wait()

        def half_step(j, b):
            ab = (b + _NBUF - 1) % _NBUF
            ob = (b + _NBUF - 2) % _NBUF
            wait_gather(b)
            start_add(j, b)

            @pl.when(j >= 1)
            def _():
                wait_add(ab)
                start_out(j - 1, ab)

            @pl.when(jnp.logical_and(j >= 2, j + 3 < nj))
            def _():
                wait_out(ob)

            @pl.when(j + 3 < nj)
            def _():
                start_gather(j + 3, ob)

        for b in range(3):
            start_gather(b, b)

        def step(i, carry):
            for b in range(_NBUF):
                half_step(i * _NBUF + b, b)
            return carry

        lax.fori_loop(0, nfull // _NBUF, step, 0)
        for j in range(nfull, nj):
            half_step(j, j % _NBUF)
        lb = (nj - 1) % _NBUF
        wait_add(lb)
        start_out(nj - 1, lb)
        for j in range(nj - _NBUF, nj):
            wait_out(j % _NBUF)

    return sc_gather, nw, nj


def _ln_body(x_ref, w_ref, b_ref, o_ref):
    x = x_ref[...]
    mu = jnp.mean(x, axis=-1, keepdims=True)
    d = x - mu
    var = jnp.mean(d * d, axis=-1, keepdims=True)
    inv = lax.rsqrt(var + _EPS)
    o_ref[...] = d * inv * w_ref[0][None, :] + b_ref[0][None, :]


def kernel(token_ids, segment_ids, word_emb, pos_emb, seg_emb, ln_w, ln_b):
    bsz, seq = token_ids.shape
    vocab, hid = word_emb.shape
    nseg = seg_emb.shape[0]
    tok = bsz * seq
    nps = nseg * seq

    info = plsc.get_sparse_core_info()
    nc, ns = info.num_cores, info.num_subcores
    sc_gather, nw, nj = _build_sc_gather(vocab, hid, tok, nps, nc, ns)

    idx = token_ids.astype(jnp.int32).reshape(nw, nj, _K)
    ps = (pos_emb[:seq][None, :, :] + seg_emb[:, None, :]).reshape(nps, hid)
    psidx = (
        segment_ids.astype(jnp.int32) * seq + jnp.arange(seq, dtype=jnp.int32)[None, :]
    ).reshape(nw, nj, _K)
    summed = sc_gather(word_emb, idx, psidx, ps)

    tb = 20480
    assert tok % tb == 0
    out = pl.pallas_call(
        _ln_body,
        grid=(tok // tb,),
        in_specs=[
            pl.BlockSpec((tb, hid), lambda i: (i, 0)),
            pl.BlockSpec((1, hid), lambda i: (0, 0)),
            pl.BlockSpec((1, hid), lambda i: (0, 0)),
        ],
        out_specs=pl.BlockSpec((tb, hid), lambda i: (i, 0)),
        out_shape=jax.ShapeDtypeStruct((tok, hid), jnp.float32),
        compiler_params=pltpu.CompilerParams(dimension_semantics=("parallel",)),
    )(summed, ln_w.reshape(1, hid), ln_b.reshape(1, hid))
    return out.reshape(bsz, seq, hid)
